# baseline (speedup 1.0000x reference)
import functools

import jax
import jax.numpy as jnp
from jax import lax
from jax.experimental import pallas as pl
from jax.experimental.pallas import tpu as pltpu
from jax.experimental.pallas import tpu_sc as plsc

VOC = 1000000
D = 64
NB, NP = 4096, 200
NC, NS, L = 2, 16, 16
NW = NC * NS
BLK = 128
NFULL = VOC // BLK
TAIL0 = NFULL * BLK
SLOTS1 = 246
SLOTS2 = 201

_mesh = dict(core_axis_name="c", subcore_axis_name="s")


def _wid():
    return lax.axis_index("s") * NC + lax.axis_index("c")


def _repack_body(tt_hbm, scr_hbm, slab0, slab1, dst0, dst1, tail_v,
                 rsem0, rsem1, wsem0, wsem1):
    w = _wid()
    slabs = (slab0, slab1)
    dsts = (dst0, dst1)
    rsems = (rsem0, rsem1)
    wsems = (wsem0, wsem1)

    def blk_of(s):
        b = w + NW * s
        return jnp.where(b < NFULL, b, w)

    def fire_read(s, b):
        off = pl.multiple_of(blk_of(s) * BLK, BLK)
        pltpu.async_copy(tt_hbm.at[:, pl.ds(off, BLK)], slabs[b], rsems[b])

    fire_read(0, 0)
    fire_read(1, 1)

    @pl.loop(0, SLOTS1, step=2)
    def _outer(g):
        for b in range(2):
            s = g + b
            pltpu.make_async_copy(
                tt_hbm.at[:, pl.ds(0, BLK)], slabs[b], rsems[b]).wait()

            @pl.when(s >= 2)
            def _drain_w():
                pltpu.make_async_copy(
                    dsts[b], scr_hbm.at[pl.ds(0, D), :], wsems[b]).wait()

            slab = slabs[b]
            dst = dsts[b]

            @plsc.parallel_loop(0, D, step=1, unroll=4)
            def _t(r2):
                for p01 in range(2):
                    col = jnp.full((L,), 2 * r2 + p01, jnp.int32)
                    for k4 in range(D // L):
                        i0 = lax.iota(jnp.int32, L) + k4 * L
                        vals = plsc.load_gather(slab, [i0, col])
                        dst[r2, pl.ds(p01 * D + k4 * L, L)] = vals * 8.0

            roff = pl.multiple_of(blk_of(s) * (BLK // 2), BLK // 2)
            pltpu.async_copy(dsts[b], scr_hbm.at[pl.ds(roff, D), :], wsems[b])

            @pl.when(s + 2 < SLOTS1)
            def _next():
                fire_read(s + 2, b)

    for b in range(2):
        pltpu.make_async_copy(
            dsts[b], scr_hbm.at[pl.ds(0, D), :], wsems[b]).wait()

    @pl.when(w == NW - 1)
    def _tail():
        ntail = VOC - TAIL0
        for j in range(D):
            pltpu.sync_copy(tt_hbm.at[j, pl.ds(TAIL0, ntail)],
                            tail_v.at[j])

        @plsc.parallel_loop(0, ntail // 2, step=1, unroll=4)
        def _tt(r2):
            for p01 in range(2):
                col = jnp.full((L,), 2 * r2 + p01, jnp.int32)
                for k4 in range(D // L):
                    i0 = lax.iota(jnp.int32, L) + k4 * L
                    vals = plsc.load_gather(tail_v, [i0, col])
                    dst0[r2, pl.ds(p01 * D + k4 * L, L)] = vals * 8.0

        pltpu.sync_copy(dst0.at[pl.ds(0, ntail // 2), :],
                        scr_hbm.at[pl.ds(TAIL0 // 2, ntail // 2), :])


def _gather_body(scr_hbm, xt_hbm, out_hbm, idx_v, r0, r1, r2v, q0, q1, q2,
                 d0, d1, gsem0, gsem1, gsem2, osem0, osem1):
    w = _wid()
    rows = (r0, r1, r2v)
    qs = (q0, q1, q2)
    gsems = (gsem0, gsem1, gsem2)
    dsts = (d0, d1)
    osems = (osem0, osem1)

    woff = pl.multiple_of(w * 128, 128)
    pltpu.sync_copy(xt_hbm.at[:, pl.ds(woff, 128)], idx_v)

    def p_of(s):
        return jnp.where(s < NP, s, NP - 1)

    def fire_gather(s, b):
        p = p_of(s)
        for k in range(128 // L):
            piece = idx_v[p, pl.ds(k * L, L)]
            qs[b][pl.ds(k * L, L)] = lax.shift_right_logical(piece, 1)
        pltpu.async_copy(scr_hbm.at[qs[b]], rows[b], gsems[b])

    fire_gather(0, 0)
    fire_gather(1, 1)
    fire_gather(2, 2)

    @pl.loop(0, SLOTS2, step=3)
    def _outer(g):
        for b in range(3):
            s = g + b
            p = p_of(s)
            d = b % 2
            pltpu.make_async_copy(
                scr_hbm.at[qs[b]], rows[b], gsems[b]).wait()

            @pl.when(s >= 2)
            def _drain_o():
                pltpu.make_async_copy(
                    dsts[d], out_hbm.at[0, :, pl.ds(0, 128)], osems[d]).wait()

            pars = []
            for k in range(128 // L):
                piece = idx_v[p, pl.ds(k * L, L)]
                pars.append(lax.shift_left(piece & 1, 6))
            row = rows[b]
            dst = dsts[d]

            @plsc.parallel_loop(0, D, step=1, unroll=4)
            def _t(j):
                for k in range(128 // L):
                    i0 = lax.iota(jnp.int32, L) + k * L
                    vals = plsc.load_gather(row, [i0, pars[k] + j])
                    dst[j, pl.ds(k * L, L)] = vals

            pltpu.async_copy(
                dst, out_hbm.at[p, :, pl.ds(woff, 128)], osems[d])

            @pl.when(s + 3 < SLOTS2)
            def _next():
                fire_gather(s + 3, b)

    for d in range(2):
        pltpu.make_async_copy(
            dsts[d], out_hbm.at[0, :, pl.ds(0, 128)], osems[d]).wait()


def _make_k1():
    return pl.kernel(
        _repack_body,
        out_type=jax.ShapeDtypeStruct((VOC // 2, 128), jnp.float32),
        mesh=plsc.VectorSubcoreMesh(**_mesh),
        scratch_types=[
            pltpu.VMEM((D, BLK), jnp.float32),
            pltpu.VMEM((D, BLK), jnp.float32),
            pltpu.VMEM((D, BLK), jnp.float32),
            pltpu.VMEM((D, BLK), jnp.float32),
            pltpu.VMEM((D, D), jnp.float32),
            pltpu.SemaphoreType.DMA,
            pltpu.SemaphoreType.DMA,
            pltpu.SemaphoreType.DMA,
            pltpu.SemaphoreType.DMA,
        ],
        compiler_params=pltpu.CompilerParams(
            use_tc_tiling_on_sc=True, needs_layout_passes=False),
    )


def _make_k2():
    return pl.kernel(
        _gather_body,
        out_type=jax.ShapeDtypeStruct((NP, D, NB), jnp.float32),
        mesh=plsc.VectorSubcoreMesh(**_mesh),
        scratch_types=[
            pltpu.VMEM((NP, 128), jnp.int32),
            pltpu.VMEM((128, 128), jnp.float32),
            pltpu.VMEM((128, 128), jnp.float32),
            pltpu.VMEM((128, 128), jnp.float32),
            pltpu.VMEM((128,), jnp.int32),
            pltpu.VMEM((128,), jnp.int32),
            pltpu.VMEM((128,), jnp.int32),
            pltpu.VMEM((D, 128), jnp.float32),
            pltpu.VMEM((D, 128), jnp.float32),
            pltpu.SemaphoreType.DMA,
            pltpu.SemaphoreType.DMA,
            pltpu.SemaphoreType.DMA,
            pltpu.SemaphoreType.DMA,
            pltpu.SemaphoreType.DMA,
        ],
        compiler_params=pltpu.CompilerParams(
            use_tc_tiling_on_sc=True, needs_layout_passes=False),
    )


@jax.jit
def kernel(x, table):
    xt = x.astype(jnp.int32).T
    tt = table.T
    scr = _make_k1()(tt)
    out3 = _make_k2()(scr, xt)
    return out3.transpose(2, 0, 1)
